# baseline (speedup 1.0000x reference)
import functools

import jax
import jax.numpy as jnp
from jax import lax
from jax.experimental import pallas as pl
from jax.experimental.pallas import tpu as pltpu
from jax.experimental.pallas import tpu_sc as plsc

_B = 4096
_HW = 225
_NCH = 12
_SROW = _NCH * _HW
_TROWS = 2380

_info = plsc.get_sparse_core_info()
_NC = _info.num_cores
_NS = _info.num_subcores
_NW = _NC * _NS
_SPW = _B // _NW
_NB = 16
_NG = _SPW // _NB

_f32 = jnp.float32
_i32 = jnp.int32


def _body(pcode_hbm, board_hbm, table_hbm, coef_hbm, val_hbm, pol_hbm,
          table_v, coef_v, pc0, pc1, bd0, bd1, po0, po1, val_v,
          sp0, sp1, sb0, sb1, so0, so1):
    cid = lax.axis_index("c")
    sid = lax.axis_index("s")
    wid = sid * _NC + cid
    base = wid * _SPW
    pcs = (pc0, pc1)
    bds = (bd0, bd1)
    pols = (po0, po1)
    sps = (sp0, sp1)
    sbs = (sb0, sb1)
    sos = (so0, so1)

    def start_in(g, par):
        s0 = base + g * _NB
        pltpu.async_copy(
            pcode_hbm.at[pl.ds(s0 * _SROW, _NB * _SROW)], pcs[par], sps[par])
        pltpu.async_copy(
            board_hbm.at[pl.ds(s0 * 2 * _HW, _NB * 2 * _HW)], bds[par], sbs[par])

    def wait_in(g, par):
        s0 = base + g * _NB
        pltpu.make_async_copy(
            pcode_hbm.at[pl.ds(s0 * _SROW, _NB * _SROW)], pcs[par], sps[par]).wait()
        pltpu.make_async_copy(
            board_hbm.at[pl.ds(s0 * 2 * _HW, _NB * 2 * _HW)], bds[par], sbs[par]).wait()

    start_in(0, 0)
    start_in(1, 1)
    pltpu.sync_copy(table_hbm, table_v)
    pltpu.sync_copy(coef_hbm, coef_v)
    c0 = coef_v[0, :]
    c1 = coef_v[1, :]
    lanes = lax.broadcasted_iota(_i32, (16,), 0)
    zero = jnp.zeros((16,), _f32)

    def pair(h, carry):
        for par in range(2):
            g = 2 * h + par
            pcode_v = pcs[par]
            board_v = bds[par]
            pol_v = pols[par]
            wait_in(g, par)

            @pl.when(h >= 1)
            def _():
                pltpu.make_async_copy(
                    pol_v, pol_hbm.at[pl.ds((base + (g - 2) * _NB) * _HW, _NB * _HW)],
                    sos[par]).wait()

            def samp(s, carry):
                winv, drawv = carry
                bv0 = s * _SROW + 10 * _HW + lanes
                bv1 = bv0 + _HW
                ov0 = s * (2 * _HW) + lanes
                ov1 = ov0 + _HW
                pb = s * _HW + lanes
                acc_w = zero
                acc_d = zero
                for j in range(14):
                    b0 = plsc.load_gather(board_v, [ov0 + 16 * j])
                    b1 = plsc.load_gather(board_v, [ov1 + 16 * j])
                    occ = (b0 + b1) > 0.0
                    i0 = plsc.load_gather(pcode_v, [bv0 + 16 * j]) * 3
                    i1 = plsc.load_gather(pcode_v, [bv1 + 16 * j]) * 3
                    p0 = plsc.load_gather(table_v, [i0])
                    v01 = plsc.load_gather(table_v, [i0 + 1])
                    v02 = plsc.load_gather(table_v, [i0 + 2])
                    p1 = plsc.load_gather(table_v, [i1])
                    v11 = plsc.load_gather(table_v, [i1 + 1])
                    v12 = plsc.load_gather(table_v, [i1 + 2])
                    acc_w = acc_w + jnp.where(occ, zero, v01 - v11)
                    acc_d = acc_d + jnp.where(occ, zero, v02 + v12)
                    pol = jnp.where(occ, zero, p0) * c0 + jnp.where(occ, zero, p1) * c1
                    plsc.store_scatter(pol_v, [pb + 16 * j], pol)
                win = jnp.sum(acc_w)
                draw = jnp.sum(acc_d)
                here = lanes == s
                winv = jnp.where(here, jnp.broadcast_to(win, (16,)), winv)
                drawv = jnp.where(here, jnp.broadcast_to(draw, (16,)), drawv)
                return winv, drawv

            winv, drawv = lax.fori_loop(0, _NB, samp, (zero, zero))

            ft0 = lanes * _SROW + 10 * _HW + 224
            ft1 = ft0 + _HW
            ob0 = lanes * (2 * _HW) + 224
            ob1 = ob0 + _HW
            b0 = plsc.load_gather(board_v, [ob0])
            b1 = plsc.load_gather(board_v, [ob1])
            occ = (b0 + b1) > 0.0
            i0 = plsc.load_gather(pcode_v, [ft0]) * 3
            i1 = plsc.load_gather(pcode_v, [ft1]) * 3
            p0 = plsc.load_gather(table_v, [i0])
            v01 = plsc.load_gather(table_v, [i0 + 1])
            v02 = plsc.load_gather(table_v, [i0 + 2])
            p1 = plsc.load_gather(table_v, [i1])
            v11 = plsc.load_gather(table_v, [i1 + 1])
            v12 = plsc.load_gather(table_v, [i1 + 2])
            winv = winv + jnp.where(occ, zero, v01 - v11)
            drawv = drawv + jnp.where(occ, zero, v02 + v12)
            polt = jnp.where(occ, zero, p0) * c0 + jnp.where(occ, zero, p1) * c1
            plsc.store_scatter(pol_v, [lanes * _HW + 224], polt)

            idx3 = (g * _NB + lanes) * 3
            plsc.store_scatter(val_v, [idx3], winv)
            plsc.store_scatter(val_v, [idx3 + 1], -winv)
            plsc.store_scatter(val_v, [idx3 + 2], drawv)

            @pl.when(h < (_NG // 2) - 1)
            def _():
                start_in(g + 2, par)

            pltpu.async_copy(
                pol_v, pol_hbm.at[pl.ds((base + g * _NB) * _HW, _NB * _HW)], sos[par])
        return 0

    lax.fori_loop(0, _NG // 2, pair, 0)
    for par in range(2):
        g = _NG - 2 + par
        pltpu.make_async_copy(
            pols[par], pol_hbm.at[pl.ds((base + g * _NB) * _HW, _NB * _HW)],
            sos[par]).wait()
    pltpu.sync_copy(val_v, val_hbm.at[pl.ds(base * 3, _SPW * 3)])


@jax.jit
def _run(pcode, board, table_flat, coef2):
    kfn = functools.partial(
        pl.kernel,
        out_type=[
            jax.ShapeDtypeStruct((_B * 3,), _f32),
            jax.ShapeDtypeStruct((_B * _HW,), _f32),
        ],
        mesh=plsc.VectorSubcoreMesh(core_axis_name="c", subcore_axis_name="s"),
        compiler_params=pltpu.CompilerParams(needs_layout_passes=False),
        scratch_types=[
            pltpu.VMEM((_TROWS * 3,), _f32),
            pltpu.VMEM((2, 16), _f32),
            pltpu.VMEM((_NB * _SROW,), _i32),
            pltpu.VMEM((_NB * _SROW,), _i32),
            pltpu.VMEM((_NB * 2 * _HW,), _f32),
            pltpu.VMEM((_NB * 2 * _HW,), _f32),
            pltpu.VMEM((_NB * _HW,), _f32),
            pltpu.VMEM((_NB * _HW,), _f32),
            pltpu.VMEM((_SPW * 3,), _f32),
            pltpu.SemaphoreType.DMA,
            pltpu.SemaphoreType.DMA,
            pltpu.SemaphoreType.DMA,
            pltpu.SemaphoreType.DMA,
            pltpu.SemaphoreType.DMA,
            pltpu.SemaphoreType.DMA,
        ],
    )(_body)
    return kfn(pcode, board, table_flat, coef2)


def kernel(sparse_feature_input, sparse_feature_dim, board_input, table, policy_stm_coef):
    del sparse_feature_dim
    pcode = sparse_feature_input.reshape(_B * _NCH * _HW)
    board = board_input.reshape(_B * 2 * _HW)
    table_flat = table.reshape(_TROWS * 3)
    coef2 = jnp.broadcast_to(policy_stm_coef.reshape(2, 1), (2, 16))
    value, policy = _run(pcode, board, table_flat, coef2)
    return value.reshape(_B, 3), policy.reshape(_B, 15, 15)


# speedup vs baseline: 2.3576x; 2.3576x over previous
import functools

import jax
import jax.numpy as jnp
from jax import lax
from jax.experimental import pallas as pl
from jax.experimental.pallas import tpu as pltpu
from jax.experimental.pallas import tpu_sc as plsc

_B = 4096
_HW = 225
_SROW = 2 * _HW
_TROWS = 2380

_info = plsc.get_sparse_core_info()
_NC = _info.num_cores
_NS = _info.num_subcores
_NW = _NC * _NS
_SPW = _B // _NW
_NB = 16
_NG = _SPW // _NB

_f32 = jnp.float32
_i32 = jnp.int32


def _body(pcode_hbm, board_hbm, table_hbm, coef_hbm, val_hbm, pol_hbm,
          table_v, coef_v, pc0, pc1, bd0, bd1, po0, po1, val_v,
          sp0, sp1, sb0, sb1, so0, so1):
    cid = lax.axis_index("c")
    sid = lax.axis_index("s")
    wid = sid * _NC + cid
    base = wid * _SPW
    pcs = (pc0, pc1)
    bds = (bd0, bd1)
    pols = (po0, po1)
    sps = (sp0, sp1)
    sbs = (sb0, sb1)
    sos = (so0, so1)

    def start_in(g, par):
        s0 = base + g * _NB
        pltpu.async_copy(
            pcode_hbm.at[pl.ds(s0 * _SROW, _NB * _SROW)], pcs[par], sps[par])
        pltpu.async_copy(
            board_hbm.at[pl.ds(s0 * 2 * _HW, _NB * 2 * _HW)], bds[par], sbs[par])

    def wait_in(g, par):
        s0 = base + g * _NB
        pltpu.make_async_copy(
            pcode_hbm.at[pl.ds(s0 * _SROW, _NB * _SROW)], pcs[par], sps[par]).wait()
        pltpu.make_async_copy(
            board_hbm.at[pl.ds(s0 * 2 * _HW, _NB * 2 * _HW)], bds[par], sbs[par]).wait()

    start_in(0, 0)
    start_in(1, 1)
    pltpu.sync_copy(table_hbm, table_v)
    pltpu.sync_copy(coef_hbm, coef_v)
    c0 = coef_v[0, :]
    c1 = coef_v[1, :]
    lanes = lax.broadcasted_iota(_i32, (16,), 0)
    zero = jnp.zeros((16,), _f32)

    def pair(h, carry):
        for par in range(2):
            g = 2 * h + par
            pcode_v = pcs[par]
            board_v = bds[par]
            pol_v = pols[par]
            wait_in(g, par)

            @pl.when(h >= 1)
            def _():
                pltpu.make_async_copy(
                    pol_v, pol_hbm.at[pl.ds((base + (g - 2) * _NB) * _HW, _NB * _HW)],
                    sos[par]).wait()

            def samp(s, carry):
                winv, drawv = carry
                bv0 = s * _SROW + lanes
                bv1 = bv0 + _HW
                ov0 = s * (2 * _HW) + lanes
                ov1 = ov0 + _HW
                pb = s * _HW + lanes
                acc_w = zero
                acc_d = zero
                for j in range(14):
                    b0 = plsc.load_gather(board_v, [ov0 + 16 * j])
                    b1 = plsc.load_gather(board_v, [ov1 + 16 * j])
                    occ = (b0 + b1) > 0.0
                    i0 = plsc.load_gather(pcode_v, [bv0 + 16 * j]) * 3
                    i1 = plsc.load_gather(pcode_v, [bv1 + 16 * j]) * 3
                    p0 = plsc.load_gather(table_v, [i0])
                    v01 = plsc.load_gather(table_v, [i0 + 1])
                    v02 = plsc.load_gather(table_v, [i0 + 2])
                    p1 = plsc.load_gather(table_v, [i1])
                    v11 = plsc.load_gather(table_v, [i1 + 1])
                    v12 = plsc.load_gather(table_v, [i1 + 2])
                    acc_w = acc_w + jnp.where(occ, zero, v01 - v11)
                    acc_d = acc_d + jnp.where(occ, zero, v02 + v12)
                    pol = jnp.where(occ, zero, p0) * c0 + jnp.where(occ, zero, p1) * c1
                    plsc.store_scatter(pol_v, [pb + 16 * j], pol)
                win = jnp.sum(acc_w)
                draw = jnp.sum(acc_d)
                here = lanes == s
                winv = jnp.where(here, jnp.broadcast_to(win, (16,)), winv)
                drawv = jnp.where(here, jnp.broadcast_to(draw, (16,)), drawv)
                return winv, drawv

            winv, drawv = lax.fori_loop(0, _NB, samp, (zero, zero))

            ft0 = lanes * _SROW + 224
            ft1 = ft0 + _HW
            ob0 = lanes * (2 * _HW) + 224
            ob1 = ob0 + _HW
            b0 = plsc.load_gather(board_v, [ob0])
            b1 = plsc.load_gather(board_v, [ob1])
            occ = (b0 + b1) > 0.0
            i0 = plsc.load_gather(pcode_v, [ft0]) * 3
            i1 = plsc.load_gather(pcode_v, [ft1]) * 3
            p0 = plsc.load_gather(table_v, [i0])
            v01 = plsc.load_gather(table_v, [i0 + 1])
            v02 = plsc.load_gather(table_v, [i0 + 2])
            p1 = plsc.load_gather(table_v, [i1])
            v11 = plsc.load_gather(table_v, [i1 + 1])
            v12 = plsc.load_gather(table_v, [i1 + 2])
            winv = winv + jnp.where(occ, zero, v01 - v11)
            drawv = drawv + jnp.where(occ, zero, v02 + v12)
            polt = jnp.where(occ, zero, p0) * c0 + jnp.where(occ, zero, p1) * c1
            plsc.store_scatter(pol_v, [lanes * _HW + 224], polt)

            idx3 = (g * _NB + lanes) * 3
            plsc.store_scatter(val_v, [idx3], winv)
            plsc.store_scatter(val_v, [idx3 + 1], -winv)
            plsc.store_scatter(val_v, [idx3 + 2], drawv)

            @pl.when(h < (_NG // 2) - 1)
            def _():
                start_in(g + 2, par)

            pltpu.async_copy(
                pol_v, pol_hbm.at[pl.ds((base + g * _NB) * _HW, _NB * _HW)], sos[par])
        return 0

    lax.fori_loop(0, _NG // 2, pair, 0)
    for par in range(2):
        g = _NG - 2 + par
        pltpu.make_async_copy(
            pols[par], pol_hbm.at[pl.ds((base + g * _NB) * _HW, _NB * _HW)],
            sos[par]).wait()
    pltpu.sync_copy(val_v, val_hbm.at[pl.ds(base * 3, _SPW * 3)])


@jax.jit
def _run(pcode, board, table_flat, coef2):
    kfn = functools.partial(
        pl.kernel,
        out_type=[
            jax.ShapeDtypeStruct((_B * 3,), _f32),
            jax.ShapeDtypeStruct((_B * _HW,), _f32),
        ],
        mesh=plsc.VectorSubcoreMesh(core_axis_name="c", subcore_axis_name="s"),
        compiler_params=pltpu.CompilerParams(needs_layout_passes=False),
        scratch_types=[
            pltpu.VMEM((_TROWS * 3,), _f32),
            pltpu.VMEM((2, 16), _f32),
            pltpu.VMEM((_NB * _SROW,), _i32),
            pltpu.VMEM((_NB * _SROW,), _i32),
            pltpu.VMEM((_NB * 2 * _HW,), _f32),
            pltpu.VMEM((_NB * 2 * _HW,), _f32),
            pltpu.VMEM((_NB * _HW,), _f32),
            pltpu.VMEM((_NB * _HW,), _f32),
            pltpu.VMEM((_SPW * 3,), _f32),
            pltpu.SemaphoreType.DMA,
            pltpu.SemaphoreType.DMA,
            pltpu.SemaphoreType.DMA,
            pltpu.SemaphoreType.DMA,
            pltpu.SemaphoreType.DMA,
            pltpu.SemaphoreType.DMA,
        ],
    )(_body)
    return kfn(pcode, board, table_flat, coef2)


def kernel(sparse_feature_input, sparse_feature_dim, board_input, table, policy_stm_coef):
    del sparse_feature_dim
    pcode = sparse_feature_input[:, 10:12].reshape(_B * 2 * _HW)
    board = board_input.reshape(_B * 2 * _HW)
    table_flat = table.reshape(_TROWS * 3)
    coef2 = jnp.broadcast_to(policy_stm_coef.reshape(2, 1), (2, 16))
    value, policy = _run(pcode, board, table_flat, coef2)
    return value.reshape(_B, 3), policy.reshape(_B, 15, 15)
